# baseline (speedup 1.0000x reference)
import functools

import jax
import jax.numpy as jnp
from jax import lax
from jax.experimental import pallas as pl
from jax.experimental.pallas import tpu as pltpu
from jax.experimental.pallas import tpu_sc as plsc

_NOH = 25
_NMH = 18
_TOTAL = 1000018
_OFF = _TOTAL - _NMH
_D = 64
_B = 16384
_NIDX = _B * _NOH
_NC, _NS = 2, 16
_NW = _NC * _NS
_PER_W = _NIDX // _NW
_CHUNK = 128
_NCHUNK = _PER_W // _CHUNK

_TB = 512
_H = 512


_SUP = 400
_NSUP = _PER_W // _SUP
_PIECES = ((0, 128), (128, 128), (256, 128), (384, 16))


def _sc_gather_body(emb_hbm, fc16_hbm, idx_hbm, out_emb, out_fc,
                    idx_all, row_all, rows_v, fcb_v, fcv_v,
                    ge, gf, ss):
    wid = lax.axis_index("s") * _NC + lax.axis_index("c")
    base = wid * _PER_W

    pltpu.sync_copy(idx_hbm.at[pl.ds(base, _PER_W)], idx_all)

    def rows_body(j, carry):
        off = j * 128
        for k in range(8):
            o = off + k * 16
            row_all[pl.ds(o, 16)] = lax.shift_right_logical(
                idx_all[pl.ds(o, 16)], 4)
        return carry

    lax.fori_loop(0, _PER_W // 128, rows_body, 0)

    def gather_cps(s, p):
        cps = []
        for o, ln in _PIECES:
            off = s * _SUP + o
            cps.append(pltpu.make_async_copy(
                emb_hbm.at[idx_all.at[pl.ds(off, ln)]],
                rows_v.at[p].at[pl.ds(o, ln)], ge[p]))
            cps.append(pltpu.make_async_copy(
                fc16_hbm.at[row_all.at[pl.ds(off, ln)]],
                fcb_v.at[p].at[pl.ds(o, ln)], gf[p]))
        return cps

    def rows_store_cp(s, p):
        return pltpu.make_async_copy(
            rows_v.at[p], out_emb.at[pl.ds(base + s * _SUP, _SUP)], ss[p])

    def fcv_store_cp(s, p):
        return pltpu.make_async_copy(
            fcv_v.at[p], out_fc.at[pl.ds(base + s * _SUP, _SUP)], ss[p])

    def fire(cps):
        for cp in cps:
            cp.start()

    def drain(cps):
        for cp in cps:
            cp.wait()

    def extract_fc(s, p):
        for j in range(_SUP // 16):
            iv = idx_all[pl.ds(s * _SUP + j * 16, 16)]
            loc = lax.iota(jnp.int32, 16) + j * 16
            fcv_v[p, pl.ds(j * 16, 16)] = plsc.load_gather(
                fcb_v.at[p], [loc, lax.bitwise_and(iv, 15)])

    fire(gather_cps(0, 0))

    def body(g, carry):
        for b in (0, 1):
            s = 2 * g + b
            q = 1 - b

            if b == 0:
                @pl.when(g > 0)
                def _():
                    rows_store_cp(s - 1, q).wait()
                    fcv_store_cp(s - 1, q).wait()
                fire(gather_cps(s + 1, q))
            else:
                @pl.when(g < _NSUP // 2 - 1)
                def _():
                    rows_store_cp(s - 1, q).wait()
                    fcv_store_cp(s - 1, q).wait()
                    fire(gather_cps(s + 1, q))

            ecps = gather_cps(s, b)
            drain([cp for i, cp in enumerate(ecps) if i % 2 == 0])
            rows_store_cp(s, b).start()
            drain([cp for i, cp in enumerate(ecps) if i % 2 == 1])
            extract_fc(s, b)
            fcv_store_cp(s, b).start()
        return carry

    lax.fori_loop(0, _NSUP // 2, body, 0)
    rows_store_cp(_NSUP - 2, 0).wait()
    fcv_store_cp(_NSUP - 2, 0).wait()
    rows_store_cp(_NSUP - 1, 1).wait()
    fcv_store_cp(_NSUP - 1, 1).wait()


@functools.cache
def _sc_gather():
    return pl.kernel(
        _sc_gather_body,
        out_type=(jax.ShapeDtypeStruct((_NIDX, _D), jnp.bfloat16),
                  jax.ShapeDtypeStruct((_NIDX,), jnp.float32)),
        mesh=plsc.VectorSubcoreMesh(core_axis_name="c", subcore_axis_name="s",
                                    num_cores=_NC, num_subcores=_NS),
        scratch_types=[
            pltpu.VMEM((_PER_W,), jnp.int32),
            pltpu.VMEM((_PER_W,), jnp.int32),
            pltpu.VMEM((2, _SUP, _D), jnp.bfloat16),
            pltpu.VMEM((2, _SUP, 16), jnp.float32),
            pltpu.VMEM((2, _SUP), jnp.float32),
            (pltpu.SemaphoreType.DMA, pltpu.SemaphoreType.DMA),
            (pltpu.SemaphoreType.DMA, pltpu.SemaphoreType.DMA),
            (pltpu.SemaphoreType.DMA, pltpu.SemaphoreType.DMA),
        ],
        compiler_params=pltpu.CompilerParams(use_tc_tiling_on_sc=False,
                                             needs_layout_passes=False),
    )


def _tc_body(emb_ref, mh_ref, fcv_ref, gt_ref, gfc_ref, s_ref,
             w1aug_ref, w1g_ref, b1_ref, w2_ref, b2_ref, w3_ref, b3_ref,
             w4_ref, c_ref, out_ref):
    f32 = jnp.float32
    emb_bf = emb_ref[...]
    mask = (mh_ref[...] != 0).astype(f32)

    sum_e = jnp.dot(mask, gt_ref[...], preferred_element_type=f32)

    haug = jnp.dot(emb_bf, w1aug_ref[...], preferred_element_type=f32)
    s = haug[:, _H:] + sum_e
    sq = jnp.dot(emb_bf * emb_bf, s_ref[...],
                 preferred_element_type=f32) + sum_e * sum_e
    fm_inter = 0.5 * jnp.sum(s * s - sq, axis=1, keepdims=True)

    fc_sum = (jnp.sum(fcv_ref[...], axis=1, keepdims=True)
              + jnp.sum(mask * gfc_ref[...], axis=1, keepdims=True))
    fm_y = c_ref[0, 0] + fc_sum + fm_inter

    h = haug[:, :_H] + jnp.dot(sum_e.astype(jnp.bfloat16), w1g_ref[...],
                               preferred_element_type=f32)
    h = jnp.maximum(h + b1_ref[...], 0.0).astype(jnp.bfloat16)
    h = jnp.maximum(jnp.dot(h, w2_ref[...], preferred_element_type=f32)
                    + b2_ref[...], 0.0).astype(jnp.bfloat16)
    h = jnp.maximum(jnp.dot(h, w3_ref[...], preferred_element_type=f32)
                    + b3_ref[...], 0.0)
    mlp = jnp.sum(h * w4_ref[...], axis=1, keepdims=True)

    z = fm_y + mlp
    out_ref[...] = 1.0 / (1.0 + jnp.exp(-z))


def _const(i):
    return (0, 0)


_tc_forward = pl.pallas_call(
    _tc_body,
    grid=(_B // _TB,),
    in_specs=[
        pl.BlockSpec((_TB, _NOH * _D), lambda i: (i, 0)),
        pl.BlockSpec((_TB, _NMH), lambda i: (i, 0)),
        pl.BlockSpec((_TB, _NOH), lambda i: (i, 0)),
        pl.BlockSpec((_NMH, _D), _const),
        pl.BlockSpec((1, _NMH), _const),
        pl.BlockSpec((_NOH * _D, _D), _const),
        pl.BlockSpec((_NOH * _D, _H + _D), _const),
        pl.BlockSpec((_D, _H), _const),
        pl.BlockSpec((1, _H), _const),
        pl.BlockSpec((_H, _H), _const),
        pl.BlockSpec((1, _H), _const),
        pl.BlockSpec((_H, _H), _const),
        pl.BlockSpec((1, _H), _const),
        pl.BlockSpec((1, _H), _const),
        pl.BlockSpec((1, 1), _const),
    ],
    out_specs=pl.BlockSpec((_TB, 1), lambda i: (i, 0)),
    out_shape=jax.ShapeDtypeStruct((_B, 1), jnp.float32),
)


def kernel(x, bias, fc_w, emb_w, W1, b1, W2, b2, W3, b3, W4, b4):
    bf16 = jnp.bfloat16
    idx = x[:, :_NOH].reshape(_NIDX)
    mh = x[:, _NOH:]

    fc16 = jnp.pad(fc_w.reshape(_TOTAL), (0, 14)).reshape(-1, 16)
    rows, fcv = _sc_gather()(emb_w.astype(jnp.bfloat16), fc16, idx)
    rows = rows.reshape(_B, _NOH * _D)
    fcv = fcv.reshape(_B, _NOH)

    gt = lax.slice(emb_w, (_OFF, 0), (_TOTAL, _D))
    gfc = lax.slice(fc_w, (_OFF, 0), (_TOTAL, 1)).reshape(1, _NMH)

    hp = _H - 400
    s_mat = jnp.tile(jnp.eye(_D, dtype=bf16), (_NOH, 1))
    w1a = jnp.concatenate(
        [jnp.pad(W1[:_NOH * _D], ((0, 0), (0, hp))).astype(bf16), s_mat],
        axis=1)
    w1g = jnp.pad(W1[_NOH * _D:], ((0, 0), (0, hp))).astype(bf16)
    b1p = jnp.pad(b1, (0, hp)).reshape(1, _H)
    w2p = jnp.pad(W2, ((0, hp), (0, hp))).astype(bf16)
    b2p = jnp.pad(b2, (0, hp)).reshape(1, _H)
    w3p = jnp.pad(W3, ((0, hp), (0, hp))).astype(bf16)
    b3p = jnp.pad(b3, (0, hp)).reshape(1, _H)
    w4r = jnp.pad(W4[:, 0], (0, hp)).reshape(1, _H)
    cst = (bias + b4).reshape(1, 1)

    y = _tc_forward(rows, mh, fcv, gt, gfc, s_mat,
                    w1a, w1g, b1p, w2p, b2p, w3p, b3p, w4r, cst)
    return y.reshape(_B)


# speedup vs baseline: 1.6679x; 1.6679x over previous
import functools

import jax
import jax.numpy as jnp
from jax import lax
from jax.experimental import pallas as pl
from jax.experimental.pallas import tpu as pltpu
from jax.experimental.pallas import tpu_sc as plsc

_NOH = 25
_NMH = 18
_TOTAL = 1000018
_OFF = _TOTAL - _NMH
_D = 64
_B = 16384
_NIDX = _B * _NOH
_NC, _NS = 2, 16
_NW = _NC * _NS
_PER_W = _NIDX // _NW
_CHUNK = 128
_NCHUNK = _PER_W // _CHUNK

_TB = 512
_H = 512


_SUP = 400
_NSUP = _PER_W // _SUP
_PIECES = ((0, 128), (128, 128), (256, 128), (384, 16))


def _make_sc_body(per_w, nsup):
  def _sc_gather_body(emb_hbm, fc16_hbm, idx_hbm, out_emb, out_fc,
                      idx_all, row_all, rows_v, fcb_v, fcv_v,
                      ge, gf, ss):
    wid = lax.axis_index("s") * _NC + lax.axis_index("c")
    base = wid * per_w

    pltpu.sync_copy(idx_hbm.at[pl.ds(base, per_w)], idx_all)

    def rows_body(j, carry):
        off = j * 128
        for k in range(8):
            o = off + k * 16
            row_all[pl.ds(o, 16)] = lax.shift_right_logical(
                idx_all[pl.ds(o, 16)], 4)
        return carry

    lax.fori_loop(0, per_w // 128, rows_body, 0)

    def gather_cps(s, p):
        cps = []
        for o, ln in _PIECES:
            off = s * _SUP + o
            cps.append(pltpu.make_async_copy(
                emb_hbm.at[idx_all.at[pl.ds(off, ln)]],
                rows_v.at[p].at[pl.ds(o, ln)], ge[p]))
            cps.append(pltpu.make_async_copy(
                fc16_hbm.at[row_all.at[pl.ds(off, ln)]],
                fcb_v.at[p].at[pl.ds(o, ln)], gf[p]))
        return cps

    def rows_store_cp(s, p):
        return pltpu.make_async_copy(
            rows_v.at[p], out_emb.at[pl.ds(base + s * _SUP, _SUP)], ss[p])

    def fcv_store_cp(s, p):
        return pltpu.make_async_copy(
            fcv_v.at[p], out_fc.at[pl.ds(base + s * _SUP, _SUP)], ss[p])

    def fire(cps):
        for cp in cps:
            cp.start()

    def drain(cps):
        for cp in cps:
            cp.wait()

    def extract_fc(s, p):
        for j in range(_SUP // 16):
            iv = idx_all[pl.ds(s * _SUP + j * 16, 16)]
            loc = lax.iota(jnp.int32, 16) + j * 16
            fcv_v[p, pl.ds(j * 16, 16)] = plsc.load_gather(
                fcb_v.at[p], [loc, lax.bitwise_and(iv, 15)])

    fire(gather_cps(0, 0))

    def body(g, carry):
        for b in (0, 1):
            s = 2 * g + b
            q = 1 - b

            if b == 0:
                @pl.when(g > 0)
                def _():
                    rows_store_cp(s - 1, q).wait()
                    fcv_store_cp(s - 1, q).wait()
                fire(gather_cps(s + 1, q))
            else:
                @pl.when(g < nsup // 2 - 1)
                def _():
                    rows_store_cp(s - 1, q).wait()
                    fcv_store_cp(s - 1, q).wait()
                    fire(gather_cps(s + 1, q))

            ecps = gather_cps(s, b)
            drain([cp for i, cp in enumerate(ecps) if i % 2 == 0])
            rows_store_cp(s, b).start()
            drain([cp for i, cp in enumerate(ecps) if i % 2 == 1])
            extract_fc(s, b)
            fcv_store_cp(s, b).start()
        return carry

    lax.fori_loop(0, nsup // 2, body, 0)
    rows_store_cp(nsup - 2, 0).wait()
    fcv_store_cp(nsup - 2, 0).wait()
    rows_store_cp(nsup - 1, 1).wait()
    fcv_store_cp(nsup - 1, 1).wait()

  return _sc_gather_body


@functools.cache
def _sc_gather(nb):
    nidx = nb * _NOH
    per_w = nidx // _NW
    nsup = per_w // _SUP
    return pl.kernel(
        _make_sc_body(per_w, nsup),
        out_type=(jax.ShapeDtypeStruct((nidx, _D), jnp.float32),
                  jax.ShapeDtypeStruct((nidx,), jnp.float32)),
        mesh=plsc.VectorSubcoreMesh(core_axis_name="c", subcore_axis_name="s",
                                    num_cores=_NC, num_subcores=_NS),
        scratch_types=[
            pltpu.VMEM((per_w,), jnp.int32),
            pltpu.VMEM((per_w,), jnp.int32),
            pltpu.VMEM((2, _SUP, _D), jnp.float32),
            pltpu.VMEM((2, _SUP, 16), jnp.float32),
            pltpu.VMEM((2, _SUP), jnp.float32),
            (pltpu.SemaphoreType.DMA, pltpu.SemaphoreType.DMA),
            (pltpu.SemaphoreType.DMA, pltpu.SemaphoreType.DMA),
            (pltpu.SemaphoreType.DMA, pltpu.SemaphoreType.DMA),
        ],
        compiler_params=pltpu.CompilerParams(use_tc_tiling_on_sc=False,
                                             needs_layout_passes=False),
    )


def _tc_body(emb_ref, mh_ref, fcv_ref, gt_ref, gfc_ref, s_ref,
             w1aug_ref, w1g_ref, b1_ref, w2_ref, b2_ref, w3_ref, b3_ref,
             w4_ref, c_ref, out_ref):
    f32 = jnp.float32
    emb_bf = emb_ref[...].astype(jnp.bfloat16)
    mask = (mh_ref[...] != 0).astype(f32)

    sum_e = jnp.dot(mask, gt_ref[...], preferred_element_type=f32)

    haug = jnp.dot(emb_bf, w1aug_ref[...], preferred_element_type=f32)
    s = haug[:, _H:] + sum_e
    sq = jnp.dot(emb_bf * emb_bf, s_ref[...],
                 preferred_element_type=f32) + sum_e * sum_e
    fm_inter = 0.5 * jnp.sum(s * s - sq, axis=1, keepdims=True)

    fc_sum = (jnp.sum(fcv_ref[...], axis=1, keepdims=True)
              + jnp.sum(mask * gfc_ref[...], axis=1, keepdims=True))
    fm_y = c_ref[0, 0] + fc_sum + fm_inter

    h = haug[:, :_H] + jnp.dot(sum_e.astype(jnp.bfloat16), w1g_ref[...],
                               preferred_element_type=f32)
    h = jnp.maximum(h + b1_ref[...], 0.0).astype(jnp.bfloat16)
    h = jnp.maximum(jnp.dot(h, w2_ref[...], preferred_element_type=f32)
                    + b2_ref[...], 0.0).astype(jnp.bfloat16)
    h = jnp.maximum(jnp.dot(h, w3_ref[...], preferred_element_type=f32)
                    + b3_ref[...], 0.0)
    mlp = jnp.sum(h * w4_ref[...], axis=1, keepdims=True)

    z = fm_y + mlp
    out_ref[...] = 1.0 / (1.0 + jnp.exp(-z))


def _const(i):
    return (0, 0)


@functools.cache
def _tc_forward(nb):
  return pl.pallas_call(
    _tc_body,
    grid=(nb // _TB,),
    in_specs=[
        pl.BlockSpec((_TB, _NOH * _D), lambda i: (i, 0)),
        pl.BlockSpec((_TB, _NMH), lambda i: (i, 0)),
        pl.BlockSpec((_TB, _NOH), lambda i: (i, 0)),
        pl.BlockSpec((_NMH, _D), _const),
        pl.BlockSpec((1, _NMH), _const),
        pl.BlockSpec((_NOH * _D, _D), _const),
        pl.BlockSpec((_NOH * _D, _H + _D), _const),
        pl.BlockSpec((_D, _H), _const),
        pl.BlockSpec((1, _H), _const),
        pl.BlockSpec((_H, _H), _const),
        pl.BlockSpec((1, _H), _const),
        pl.BlockSpec((_H, _H), _const),
        pl.BlockSpec((1, _H), _const),
        pl.BlockSpec((1, _H), _const),
        pl.BlockSpec((1, 1), _const),
    ],
    out_specs=pl.BlockSpec((_TB, 1), lambda i: (i, 0)),
    out_shape=jax.ShapeDtypeStruct((nb, 1), jnp.float32),
  )


_KSPLIT = 2


def kernel(x, bias, fc_w, emb_w, W1, b1, W2, b2, W3, b3, W4, b4):
    bf16 = jnp.bfloat16
    fc16 = jnp.pad(fc_w.reshape(_TOTAL), (0, 14)).reshape(-1, 16)
    gt = lax.slice(emb_w, (_OFF, 0), (_TOTAL, _D))
    gfc = lax.slice(fc_w, (_OFF, 0), (_TOTAL, 1)).reshape(1, _NMH)

    hp = _H - 400
    s_mat = jnp.tile(jnp.eye(_D, dtype=bf16), (_NOH, 1))
    w1a = jnp.concatenate(
        [jnp.pad(W1[:_NOH * _D], ((0, 0), (0, hp))).astype(bf16), s_mat],
        axis=1)
    w1g = jnp.pad(W1[_NOH * _D:], ((0, 0), (0, hp))).astype(bf16)
    b1p = jnp.pad(b1, (0, hp)).reshape(1, _H)
    w2p = jnp.pad(W2, ((0, hp), (0, hp))).astype(bf16)
    b2p = jnp.pad(b2, (0, hp)).reshape(1, _H)
    w3p = jnp.pad(W3, ((0, hp), (0, hp))).astype(bf16)
    b3p = jnp.pad(b3, (0, hp)).reshape(1, _H)
    w4r = jnp.pad(W4[:, 0], (0, hp)).reshape(1, _H)
    cst = (bias + b4).reshape(1, 1)

    nb = _B // _KSPLIT
    ys = []
    for h in range(_KSPLIT):
        xh = lax.slice(x, (h * nb, 0), ((h + 1) * nb, x.shape[1]))
        idx = xh[:, :_NOH].reshape(nb * _NOH)
        mh = xh[:, _NOH:]
        rows, fcv = _sc_gather(nb)(emb_w, fc16, idx)
        y = _tc_forward(nb)(rows.reshape(nb, _NOH * _D), mh,
                            fcv.reshape(nb, _NOH), gt, gfc, s_mat,
                            w1a, w1g, b1p, w2p, b2p, w3p, b3p, w4r, cst)
        ys.append(y)
    return jnp.concatenate(ys, axis=0).reshape(_B)


# speedup vs baseline: 1.6705x; 1.0015x over previous
import functools

import jax
import jax.numpy as jnp
from jax import lax
from jax.experimental import pallas as pl
from jax.experimental.pallas import tpu as pltpu
from jax.experimental.pallas import tpu_sc as plsc

_NOH = 25
_NMH = 18
_TOTAL = 1000018
_OFF = _TOTAL - _NMH
_D = 64
_B = 16384
_NIDX = _B * _NOH
_NC, _NS = 2, 16
_NW = _NC * _NS
_PER_W = _NIDX // _NW
_CHUNK = 128
_NCHUNK = _PER_W // _CHUNK

_TB = 1024
_H = 512


_SUP = 400
_NSUP = _PER_W // _SUP
_PIECES = ((0, 128), (128, 128), (256, 128), (384, 16))


def _make_sc_body(per_w, nsup):
  def _sc_gather_body(emb_hbm, fc16_hbm, idx_hbm, out_emb, out_fc,
                      idx_all, row_all, rows_v, fcb_v, fcv_v,
                      ge, gf, ss):
    wid = lax.axis_index("s") * _NC + lax.axis_index("c")
    base = wid * per_w

    pltpu.sync_copy(idx_hbm.at[pl.ds(base, per_w)], idx_all)

    def rows_body(j, carry):
        off = j * 128
        for k in range(8):
            o = off + k * 16
            row_all[pl.ds(o, 16)] = lax.shift_right_logical(
                idx_all[pl.ds(o, 16)], 4)
        return carry

    lax.fori_loop(0, per_w // 128, rows_body, 0)

    def gather_cps(s, p):
        cps = []
        for o, ln in _PIECES:
            off = s * _SUP + o
            cps.append(pltpu.make_async_copy(
                emb_hbm.at[idx_all.at[pl.ds(off, ln)]],
                rows_v.at[p].at[pl.ds(o, ln)], ge[p]))
            cps.append(pltpu.make_async_copy(
                fc16_hbm.at[row_all.at[pl.ds(off, ln)]],
                fcb_v.at[p].at[pl.ds(o, ln)], gf[p]))
        return cps

    def rows_store_cp(s, p):
        return pltpu.make_async_copy(
            rows_v.at[p], out_emb.at[pl.ds(base + s * _SUP, _SUP)], ss[p])

    def fcv_store_cp(s, p):
        return pltpu.make_async_copy(
            fcv_v.at[p], out_fc.at[pl.ds(base + s * _SUP, _SUP)], ss[p])

    def fire(cps):
        for cp in cps:
            cp.start()

    def drain(cps):
        for cp in cps:
            cp.wait()

    def extract_fc(s, p):
        for j in range(_SUP // 16):
            iv = idx_all[pl.ds(s * _SUP + j * 16, 16)]
            loc = lax.iota(jnp.int32, 16) + j * 16
            fcv_v[p, pl.ds(j * 16, 16)] = plsc.load_gather(
                fcb_v.at[p], [loc, lax.bitwise_and(iv, 15)])

    fire(gather_cps(0, 0))

    def body(g, carry):
        for b in (0, 1):
            s = 2 * g + b
            q = 1 - b

            if b == 0:
                @pl.when(g > 0)
                def _():
                    rows_store_cp(s - 1, q).wait()
                    fcv_store_cp(s - 1, q).wait()
                fire(gather_cps(s + 1, q))
            else:
                @pl.when(g < nsup // 2 - 1)
                def _():
                    rows_store_cp(s - 1, q).wait()
                    fcv_store_cp(s - 1, q).wait()
                    fire(gather_cps(s + 1, q))

            ecps = gather_cps(s, b)
            drain([cp for i, cp in enumerate(ecps) if i % 2 == 0])
            rows_store_cp(s, b).start()
            drain([cp for i, cp in enumerate(ecps) if i % 2 == 1])
            extract_fc(s, b)
            fcv_store_cp(s, b).start()
        return carry

    lax.fori_loop(0, nsup // 2, body, 0)
    rows_store_cp(nsup - 2, 0).wait()
    fcv_store_cp(nsup - 2, 0).wait()
    rows_store_cp(nsup - 1, 1).wait()
    fcv_store_cp(nsup - 1, 1).wait()

  return _sc_gather_body


@functools.cache
def _sc_gather(nb):
    nidx = nb * _NOH
    per_w = nidx // _NW
    nsup = per_w // _SUP
    return pl.kernel(
        _make_sc_body(per_w, nsup),
        out_type=(jax.ShapeDtypeStruct((nidx, _D), jnp.float32),
                  jax.ShapeDtypeStruct((nidx,), jnp.float32)),
        mesh=plsc.VectorSubcoreMesh(core_axis_name="c", subcore_axis_name="s",
                                    num_cores=_NC, num_subcores=_NS),
        scratch_types=[
            pltpu.VMEM((per_w,), jnp.int32),
            pltpu.VMEM((per_w,), jnp.int32),
            pltpu.VMEM((2, _SUP, _D), jnp.float32),
            pltpu.VMEM((2, _SUP, 16), jnp.float32),
            pltpu.VMEM((2, _SUP), jnp.float32),
            (pltpu.SemaphoreType.DMA, pltpu.SemaphoreType.DMA),
            (pltpu.SemaphoreType.DMA, pltpu.SemaphoreType.DMA),
            (pltpu.SemaphoreType.DMA, pltpu.SemaphoreType.DMA),
        ],
        compiler_params=pltpu.CompilerParams(use_tc_tiling_on_sc=False,
                                             needs_layout_passes=False),
    )


def _tc_body(emb_ref, mh_ref, fcv_ref, gt_ref, gfc_ref, s_ref,
             w1aug_ref, w1g_ref, b1_ref, w2_ref, b2_ref, w3_ref, b3_ref,
             w4_ref, c_ref, out_ref):
    f32 = jnp.float32
    emb_bf = emb_ref[...].astype(jnp.bfloat16)
    mask = (mh_ref[...] != 0).astype(f32)

    sum_e = jnp.dot(mask, gt_ref[...], preferred_element_type=f32)

    haug = jnp.dot(emb_bf, w1aug_ref[...], preferred_element_type=f32)
    s = haug[:, _H:] + sum_e
    sq = jnp.dot(emb_bf * emb_bf, s_ref[...],
                 preferred_element_type=f32) + sum_e * sum_e
    fm_inter = 0.5 * jnp.sum(s * s - sq, axis=1, keepdims=True)

    fc_sum = (jnp.sum(fcv_ref[...], axis=1, keepdims=True)
              + jnp.sum(mask * gfc_ref[...], axis=1, keepdims=True))
    fm_y = c_ref[0, 0] + fc_sum + fm_inter

    h = haug[:, :_H] + jnp.dot(sum_e.astype(jnp.bfloat16), w1g_ref[...],
                               preferred_element_type=f32)
    h = jnp.maximum(h + b1_ref[...], 0.0).astype(jnp.bfloat16)
    h = jnp.maximum(jnp.dot(h, w2_ref[...], preferred_element_type=f32)
                    + b2_ref[...], 0.0).astype(jnp.bfloat16)
    h = jnp.maximum(jnp.dot(h, w3_ref[...], preferred_element_type=f32)
                    + b3_ref[...], 0.0)
    mlp = jnp.sum(h * w4_ref[...], axis=1, keepdims=True)

    z = fm_y + mlp
    out_ref[...] = 1.0 / (1.0 + jnp.exp(-z))


def _const(i):
    return (0, 0)


@functools.cache
def _tc_forward(nb):
  return pl.pallas_call(
    _tc_body,
    grid=(nb // _TB,),
    in_specs=[
        pl.BlockSpec((_TB, _NOH * _D), lambda i: (i, 0)),
        pl.BlockSpec((_TB, _NMH), lambda i: (i, 0)),
        pl.BlockSpec((_TB, _NOH), lambda i: (i, 0)),
        pl.BlockSpec((_NMH, _D), _const),
        pl.BlockSpec((1, _NMH), _const),
        pl.BlockSpec((_NOH * _D, _D), _const),
        pl.BlockSpec((_NOH * _D, _H + _D), _const),
        pl.BlockSpec((_D, _H), _const),
        pl.BlockSpec((1, _H), _const),
        pl.BlockSpec((_H, _H), _const),
        pl.BlockSpec((1, _H), _const),
        pl.BlockSpec((_H, _H), _const),
        pl.BlockSpec((1, _H), _const),
        pl.BlockSpec((1, _H), _const),
        pl.BlockSpec((1, 1), _const),
    ],
    out_specs=pl.BlockSpec((_TB, 1), lambda i: (i, 0)),
    out_shape=jax.ShapeDtypeStruct((nb, 1), jnp.float32),
  )


_KSPLIT = 2


def kernel(x, bias, fc_w, emb_w, W1, b1, W2, b2, W3, b3, W4, b4):
    bf16 = jnp.bfloat16
    fc16 = jnp.pad(fc_w.reshape(_TOTAL), (0, 14)).reshape(-1, 16)
    gt = lax.slice(emb_w, (_OFF, 0), (_TOTAL, _D))
    gfc = lax.slice(fc_w, (_OFF, 0), (_TOTAL, 1)).reshape(1, _NMH)

    hp = _H - 400
    s_mat = jnp.tile(jnp.eye(_D, dtype=bf16), (_NOH, 1))
    w1a = jnp.concatenate(
        [jnp.pad(W1[:_NOH * _D], ((0, 0), (0, hp))).astype(bf16), s_mat],
        axis=1)
    w1g = jnp.pad(W1[_NOH * _D:], ((0, 0), (0, hp))).astype(bf16)
    b1p = jnp.pad(b1, (0, hp)).reshape(1, _H)
    w2p = jnp.pad(W2, ((0, hp), (0, hp))).astype(bf16)
    b2p = jnp.pad(b2, (0, hp)).reshape(1, _H)
    w3p = jnp.pad(W3, ((0, hp), (0, hp))).astype(bf16)
    b3p = jnp.pad(b3, (0, hp)).reshape(1, _H)
    w4r = jnp.pad(W4[:, 0], (0, hp)).reshape(1, _H)
    cst = (bias + b4).reshape(1, 1)

    nb = _B // _KSPLIT
    ys = []
    for h in range(_KSPLIT):
        xh = lax.slice(x, (h * nb, 0), ((h + 1) * nb, x.shape[1]))
        idx = xh[:, :_NOH].reshape(nb * _NOH)
        mh = xh[:, _NOH:]
        rows, fcv = _sc_gather(nb)(emb_w, fc16, idx)
        y = _tc_forward(nb)(rows.reshape(nb, _NOH * _D), mh,
                            fcv.reshape(nb, _NOH), gt, gfc, s_mat,
                            w1a, w1g, b1p, w2p, b2p, w3p, b3p, w4r, cst)
        ys.append(y)
    return jnp.concatenate(ys, axis=0).reshape(_B)


# speedup vs baseline: 1.6717x; 1.0007x over previous
import functools

import jax
import jax.numpy as jnp
from jax import lax
from jax.experimental import pallas as pl
from jax.experimental.pallas import tpu as pltpu
from jax.experimental.pallas import tpu_sc as plsc

_NOH = 25
_NMH = 18
_TOTAL = 1000018
_OFF = _TOTAL - _NMH
_D = 64
_B = 16384
_NIDX = _B * _NOH
_NC, _NS = 2, 16
_NW = _NC * _NS
_PER_W = _NIDX // _NW
_CHUNK = 128
_NCHUNK = _PER_W // _CHUNK

_TB = 1024
_H = 512


_SUP = 640
_PIECES = tuple((o, 128) for o in range(0, _SUP, 128))


def _make_sc_body(per_w, nsup):
  def _sc_gather_body(emb_hbm, fc16_hbm, idx_hbm, out_emb, out_fc,
                      idx_all, row_all, rows_v, fcb_v, fcv_v,
                      ge, gf, ss):
    wid = lax.axis_index("s") * _NC + lax.axis_index("c")
    base = wid * per_w

    pltpu.sync_copy(idx_hbm.at[pl.ds(base, per_w)], idx_all)

    def rows_body(j, carry):
        off = j * 128
        for k in range(8):
            o = off + k * 16
            row_all[pl.ds(o, 16)] = lax.shift_right_logical(
                idx_all[pl.ds(o, 16)], 4)
        return carry

    lax.fori_loop(0, per_w // 128, rows_body, 0)

    def gather_cps(s, p):
        cps = []
        for o, ln in _PIECES:
            off = s * _SUP + o
            cps.append(pltpu.make_async_copy(
                emb_hbm.at[idx_all.at[pl.ds(off, ln)]],
                rows_v.at[p].at[pl.ds(o, ln)], ge[p]))
            cps.append(pltpu.make_async_copy(
                fc16_hbm.at[row_all.at[pl.ds(off, ln)]],
                fcb_v.at[p].at[pl.ds(o, ln)], gf[p]))
        return cps

    def rows_store_cp(s, p):
        return pltpu.make_async_copy(
            rows_v.at[p], out_emb.at[pl.ds(base + s * _SUP, _SUP)], ss[p])

    def fcv_store_cp(s, p):
        return pltpu.make_async_copy(
            fcv_v.at[p], out_fc.at[pl.ds(base + s * _SUP, _SUP)], ss[p])

    def fire(cps):
        for cp in cps:
            cp.start()

    def drain(cps):
        for cp in cps:
            cp.wait()

    def extract_fc(s, p):
        for j in range(_SUP // 16):
            iv = idx_all[pl.ds(s * _SUP + j * 16, 16)]
            loc = lax.iota(jnp.int32, 16) + j * 16
            fcv_v[p, pl.ds(j * 16, 16)] = plsc.load_gather(
                fcb_v.at[p], [loc, lax.bitwise_and(iv, 15)])

    fire(gather_cps(0, 0))

    def body(g, carry):
        for b in (0, 1):
            s = 2 * g + b
            q = 1 - b

            if b == 0:
                @pl.when(g > 0)
                def _():
                    rows_store_cp(s - 1, q).wait()
                    fcv_store_cp(s - 1, q).wait()
                fire(gather_cps(s + 1, q))
            else:
                @pl.when(g < nsup // 2 - 1)
                def _():
                    rows_store_cp(s - 1, q).wait()
                    fcv_store_cp(s - 1, q).wait()
                    fire(gather_cps(s + 1, q))

            ecps = gather_cps(s, b)
            drain([cp for i, cp in enumerate(ecps) if i % 2 == 0])
            rows_store_cp(s, b).start()
            drain([cp for i, cp in enumerate(ecps) if i % 2 == 1])
            extract_fc(s, b)
            fcv_store_cp(s, b).start()
        return carry

    lax.fori_loop(0, nsup // 2, body, 0)
    rows_store_cp(nsup - 2, 0).wait()
    fcv_store_cp(nsup - 2, 0).wait()
    rows_store_cp(nsup - 1, 1).wait()
    fcv_store_cp(nsup - 1, 1).wait()

  return _sc_gather_body


@functools.cache
def _sc_gather(nb):
    nidx = nb * _NOH
    per_w = nidx // _NW
    nsup = per_w // _SUP
    return pl.kernel(
        _make_sc_body(per_w, nsup),
        out_type=(jax.ShapeDtypeStruct((nidx, _D), jnp.float32),
                  jax.ShapeDtypeStruct((nidx,), jnp.float32)),
        mesh=plsc.VectorSubcoreMesh(core_axis_name="c", subcore_axis_name="s",
                                    num_cores=_NC, num_subcores=_NS),
        scratch_types=[
            pltpu.VMEM((per_w,), jnp.int32),
            pltpu.VMEM((per_w,), jnp.int32),
            pltpu.VMEM((2, _SUP, _D), jnp.float32),
            pltpu.VMEM((2, _SUP, 16), jnp.float32),
            pltpu.VMEM((2, _SUP), jnp.float32),
            (pltpu.SemaphoreType.DMA, pltpu.SemaphoreType.DMA),
            (pltpu.SemaphoreType.DMA, pltpu.SemaphoreType.DMA),
            (pltpu.SemaphoreType.DMA, pltpu.SemaphoreType.DMA),
        ],
        compiler_params=pltpu.CompilerParams(use_tc_tiling_on_sc=False,
                                             needs_layout_passes=False),
    )


def _tc_body(emb_ref, mh_ref, fcv_ref, gt_ref, gfc_ref, s_ref,
             w1aug_ref, w1g_ref, b1_ref, w2_ref, b2_ref, w3_ref, b3_ref,
             w4_ref, c_ref, out_ref):
    f32 = jnp.float32
    emb_bf = emb_ref[...].astype(jnp.bfloat16)
    mask = (mh_ref[...] != 0).astype(f32)

    sum_e = jnp.dot(mask, gt_ref[...], preferred_element_type=f32)

    haug = jnp.dot(emb_bf, w1aug_ref[...], preferred_element_type=f32)
    s = haug[:, _H:] + sum_e
    sq = jnp.dot(emb_bf * emb_bf, s_ref[...],
                 preferred_element_type=f32) + sum_e * sum_e
    fm_inter = 0.5 * jnp.sum(s * s - sq, axis=1, keepdims=True)

    fc_sum = (jnp.sum(fcv_ref[...], axis=1, keepdims=True)
              + jnp.sum(mask * gfc_ref[...], axis=1, keepdims=True))
    fm_y = c_ref[0, 0] + fc_sum + fm_inter

    h = haug[:, :_H] + jnp.dot(sum_e.astype(jnp.bfloat16), w1g_ref[...],
                               preferred_element_type=f32)
    h = jnp.maximum(h + b1_ref[...], 0.0).astype(jnp.bfloat16)
    h = jnp.maximum(jnp.dot(h, w2_ref[...], preferred_element_type=f32)
                    + b2_ref[...], 0.0).astype(jnp.bfloat16)
    h = jnp.maximum(jnp.dot(h, w3_ref[...], preferred_element_type=f32)
                    + b3_ref[...], 0.0)
    mlp = jnp.sum(h * w4_ref[...], axis=1, keepdims=True)

    z = fm_y + mlp
    out_ref[...] = 1.0 / (1.0 + jnp.exp(-z))


def _const(i):
    return (0, 0)


@functools.cache
def _tc_forward(nb):
  return pl.pallas_call(
    _tc_body,
    grid=(nb // _TB,),
    in_specs=[
        pl.BlockSpec((_TB, _NOH * _D), lambda i: (i, 0)),
        pl.BlockSpec((_TB, _NMH), lambda i: (i, 0)),
        pl.BlockSpec((_TB, _NOH), lambda i: (i, 0)),
        pl.BlockSpec((_NMH, _D), _const),
        pl.BlockSpec((1, _NMH), _const),
        pl.BlockSpec((_NOH * _D, _D), _const),
        pl.BlockSpec((_NOH * _D, _H + _D), _const),
        pl.BlockSpec((_D, _H), _const),
        pl.BlockSpec((1, _H), _const),
        pl.BlockSpec((_H, _H), _const),
        pl.BlockSpec((1, _H), _const),
        pl.BlockSpec((_H, _H), _const),
        pl.BlockSpec((1, _H), _const),
        pl.BlockSpec((1, _H), _const),
        pl.BlockSpec((1, 1), _const),
    ],
    out_specs=pl.BlockSpec((_TB, 1), lambda i: (i, 0)),
    out_shape=jax.ShapeDtypeStruct((nb, 1), jnp.float32),
  )


_KSPLIT = 2


def kernel(x, bias, fc_w, emb_w, W1, b1, W2, b2, W3, b3, W4, b4):
    bf16 = jnp.bfloat16
    fc16 = jnp.pad(fc_w.reshape(_TOTAL), (0, 14)).reshape(-1, 16)
    gt = lax.slice(emb_w, (_OFF, 0), (_TOTAL, _D))
    gfc = lax.slice(fc_w, (_OFF, 0), (_TOTAL, 1)).reshape(1, _NMH)

    hp = _H - 400
    s_mat = jnp.tile(jnp.eye(_D, dtype=bf16), (_NOH, 1))
    w1a = jnp.concatenate(
        [jnp.pad(W1[:_NOH * _D], ((0, 0), (0, hp))).astype(bf16), s_mat],
        axis=1)
    w1g = jnp.pad(W1[_NOH * _D:], ((0, 0), (0, hp))).astype(bf16)
    b1p = jnp.pad(b1, (0, hp)).reshape(1, _H)
    w2p = jnp.pad(W2, ((0, hp), (0, hp))).astype(bf16)
    b2p = jnp.pad(b2, (0, hp)).reshape(1, _H)
    w3p = jnp.pad(W3, ((0, hp), (0, hp))).astype(bf16)
    b3p = jnp.pad(b3, (0, hp)).reshape(1, _H)
    w4r = jnp.pad(W4[:, 0], (0, hp)).reshape(1, _H)
    cst = (bias + b4).reshape(1, 1)

    nb = _B // _KSPLIT
    ys = []
    for h in range(_KSPLIT):
        xh = lax.slice(x, (h * nb, 0), ((h + 1) * nb, x.shape[1]))
        idx = xh[:, :_NOH].reshape(nb * _NOH)
        mh = xh[:, _NOH:]
        rows, fcv = _sc_gather(nb)(emb_w, fc16, idx)
        y = _tc_forward(nb)(rows.reshape(nb, _NOH * _D), mh,
                            fcv.reshape(nb, _NOH), gt, gfc, s_mat,
                            w1a, w1g, b1p, w2p, b2p, w3p, b3p, w4r, cst)
        ys.append(y)
    return jnp.concatenate(ys, axis=0).reshape(_B)
